# baseline (speedup 1.0000x reference)
import jax
import jax.numpy as jnp
from jax.experimental import pallas as pl

_DEPTH = 1000
_CBLK = 8


def _body(idxt_ref, out_ref):
    i = pl.program_id(0)
    idxt = idxt_ref[...]
    s, n = idxt.shape
    c = jax.lax.broadcasted_iota(jnp.int32, (s, _CBLK, n), 1) + i * _CBLK
    out_ref[...] = (idxt[:, None, :] == c).astype(jnp.float32)


def kernel(indices):
    idxt = indices.astype(jnp.int32).T
    s, n = idxt.shape
    out = pl.pallas_call(
        _body,
        grid=(_DEPTH // _CBLK,),
        in_specs=[pl.BlockSpec((s, n), lambda i: (0, 0))],
        out_specs=pl.BlockSpec((s, _CBLK, n), lambda i: (0, i, 0)),
        out_shape=jax.ShapeDtypeStruct((s, _DEPTH, n), jnp.float32),
    )(idxt)
    return out.transpose(2, 0, 1)


# speedup vs baseline: 1.2039x; 1.2039x over previous
import jax
import jax.numpy as jnp
from jax.experimental import pallas as pl

_DEPTH = 1000
_CBLK = 64


def _body(idxt_ref, out_ref):
    i = pl.program_id(0)
    idxt = idxt_ref[...]
    s, n = idxt.shape
    c = jax.lax.broadcasted_iota(jnp.int32, (s, _CBLK, n), 1) + i * _CBLK
    out_ref[...] = (idxt[:, None, :] == c).astype(jnp.float32)


def kernel(indices):
    idxt = indices.astype(jnp.int32).T
    s, n = idxt.shape
    out = pl.pallas_call(
        _body,
        grid=(pl.cdiv(_DEPTH, _CBLK),),
        in_specs=[pl.BlockSpec((s, n), lambda i: (0, 0))],
        out_specs=pl.BlockSpec((s, _CBLK, n), lambda i: (0, i, 0)),
        out_shape=jax.ShapeDtypeStruct((s, _DEPTH, n), jnp.float32),
    )(idxt)
    return out.transpose(2, 0, 1)
